# baseline (speedup 1.0000x reference)
import functools

import jax
import jax.numpy as jnp
from jax import lax
from jax.experimental import pallas as pl
from jax.experimental.pallas import tpu as pltpu
from jax.experimental.pallas import tpu_sc as plsc

NC = 2
NS = 16
NW = NC * NS
CHUNK = 128


NBUF = 4


def _make_sc_agg(n_pad, d, e_pad, with_deg):
  t_steps = e_pad // (NW * CHUNK)
  assert t_steps % NBUF == 0
  rowch = n_pad // CHUNK
  mesh = plsc.VectorSubcoreMesh(
      core_axis_name="c", subcore_axis_name="s",
      num_cores=NC, num_subcores=NS)

  out_type = [jax.ShapeDtypeStruct((NC, n_pad, d), jnp.float32)]
  if with_deg:
    out_type.append(jax.ShapeDtypeStruct((NC, n_pad), jnp.float32))

  scratch = [
      [pltpu.VMEM((CHUNK,), jnp.int32) for _ in range(2 * NBUF)],
      [pltpu.VMEM((CHUNK, d), jnp.float32) for _ in range(2)],
      pltpu.VMEM((CHUNK,), jnp.float32),
      pltpu.VMEM_SHARED((n_pad, d), jnp.float32),
      pltpu.VMEM_SHARED((n_pad,), jnp.float32),
      [pltpu.SemaphoreType.DMA for _ in range(NBUF)],
      [pltpu.SemaphoreType.DMA for _ in range(2)],
      pltpu.SemaphoreType.DMA,
  ]

  def body(feats, srci, dsti, z2d, ones_in, *rest):
    if with_deg:
      acc_out, deg_out = rest[0], rest[1]
      rest = rest[2:]
    else:
      acc_out = rest[0]
      deg_out = None
      rest = rest[1:]
    ibufs, rows, ones_v, acc_sh, deg_sh, isems, rsems, dsem = rest
    sbuf = ibufs[:NBUF]
    dbuf = ibufs[NBUF:]

    c = lax.axis_index("c")
    s = lax.axis_index("s")
    wid = s * NC + c

    pltpu.sync_copy(ones_in, ones_v)
    pltpu.sync_copy(z2d, rows[0])
    for jj in range((rowch + NS - 1) // NS):
      j = jj * NS + s

      @pl.when(j < rowch)
      def _():
        pltpu.sync_copy(rows[0], acc_sh.at[pl.ds(j * CHUNK, CHUNK)])
        if with_deg:
          pltpu.sync_copy(rows[0].at[0], deg_sh.at[pl.ds(j * CHUNK, CHUNK)])

    plsc.subcore_barrier()

    def idx_fetch(t, i):
      pltpu.async_copy(srci.at[wid, t], sbuf[i], isems[i])
      pltpu.async_copy(dsti.at[wid, t], dbuf[i], isems[i])

    def idx_wait(t, i):
      pltpu.make_async_copy(srci.at[wid, t], sbuf[i], isems[i]).wait()
      pltpu.make_async_copy(dsti.at[wid, t], dbuf[i], isems[i]).wait()

    def gather_start(i, r):
      pltpu.async_copy(feats.at[sbuf[i]], rows[r], rsems[r])

    def gather_wait(i, r):
      pltpu.make_async_copy(feats.at[sbuf[i]], rows[r], rsems[r]).wait()

    def deg_fire(i):
      pltpu.async_copy(ones_v, deg_sh.at[dbuf[i]], dsem, add=True)

    def deg_wait(i):
      pltpu.make_async_copy(ones_v, deg_sh.at[dbuf[i]], dsem).wait()

    idx_fetch(0, 0)
    idx_fetch(1, 1)
    idx_wait(0, 0)
    gather_start(0, 0)

    def step(k, carry):
      for b in range(NBUF):
        t = NBUF * k + b
        r = b % 2

        @pl.when(t + 2 < t_steps)
        def _():
          idx_fetch(t + 2, (b + 2) % NBUF)
        gather_wait(b, r)

        @pl.when(t + 1 < t_steps)
        def _():
          idx_wait(t + 1, (b + 1) % NBUF)
          gather_start((b + 1) % NBUF, (r + 1) % 2)
        pltpu.sync_copy(rows[r], acc_sh.at[dbuf[b]], add=True)
        if with_deg:
          @pl.when(t > 0)
          def _():
            deg_wait((b - 1) % NBUF)
          deg_fire(b)
      return carry

    lax.fori_loop(0, t_steps // NBUF, step, 0)
    if with_deg:
      deg_wait((t_steps - 1) % NBUF)

    plsc.subcore_barrier()

    for jj in range((rowch + NS - 1) // NS):
      j = jj * NS + s

      @pl.when(j < rowch)
      def _():
        pltpu.sync_copy(acc_sh.at[pl.ds(j * CHUNK, CHUNK)],
                        acc_out.at[c, pl.ds(j * CHUNK, CHUNK)])
        if with_deg:
          pltpu.sync_copy(deg_sh.at[pl.ds(j * CHUNK, CHUNK)],
                          deg_out.at[c, pl.ds(j * CHUNK, CHUNK)])

  return pl.kernel(body, out_type=out_type, mesh=mesh, scratch_types=scratch)


def _tc_layer(acc, deg3, xin, wl, wr, b, row_blk):
  n, d = xin.shape
  nb = n // row_blk

  def body(acc_ref, deg_ref, x_ref, wl_ref, wr_ref, b_ref, o_ref):
    dsl = deg_ref[0]
    dtot = jnp.maximum(dsl[0] + dsl[1], 1.0)
    agg = (acc_ref[0] + acc_ref[1]) / dtot[:, None]
    h = (jnp.dot(agg, wl_ref[...], preferred_element_type=jnp.float32)
         + jnp.dot(x_ref[...], wr_ref[...], preferred_element_type=jnp.float32)
         + b_ref[...])
    o_ref[...] = jnp.maximum(h, 0.0)

  return pl.pallas_call(
      body,
      grid=(nb,),
      in_specs=[
          pl.BlockSpec((NC, row_blk, d), lambda i: (0, i, 0)),
          pl.BlockSpec((1, NC, row_blk), lambda i: (i, 0, 0)),
          pl.BlockSpec((row_blk, d), lambda i: (i, 0)),
          pl.BlockSpec((d, d), lambda i: (0, 0)),
          pl.BlockSpec((d, d), lambda i: (0, 0)),
          pl.BlockSpec((1, d), lambda i: (0, 0)),
      ],
      out_specs=pl.BlockSpec((row_blk, d), lambda i: (i, 0)),
      out_shape=jax.ShapeDtypeStruct((n, d), jnp.float32),
  )(acc, deg3, xin, wl, wr, b)


def _tc_final(acc, deg3, h1, wl, wr, b, bat3, wlin_p, blin_p, n_graphs,
              row_blk):
  n, d = h1.shape
  nb = n // row_blk

  def body(acc_ref, deg_ref, h1_ref, wl_ref, wr_ref, b_ref, bat_ref,
           wlin_ref, blin_ref, o_ref, pool_s, cnt_s):
    i = pl.program_id(0)
    dsl = deg_ref[0]
    dtot = jnp.maximum(dsl[0] + dsl[1], 1.0)
    agg = (acc_ref[0] + acc_ref[1]) / dtot[:, None]
    h2 = jnp.maximum(
        jnp.dot(agg, wl_ref[...], preferred_element_type=jnp.float32)
        + jnp.dot(h1_ref[...], wr_ref[...], preferred_element_type=jnp.float32)
        + b_ref[...], 0.0)

    bat = bat_ref[0, 0, :]
    seg = lax.broadcasted_iota(jnp.int32, (n_graphs, row_blk), 0)
    m = (seg == bat[None, :]).astype(jnp.float32)
    p_part = jnp.dot(m, h2, preferred_element_type=jnp.float32)
    c_part = jnp.broadcast_to(
        jnp.sum(m, axis=1, keepdims=True), (n_graphs, d))

    @pl.when(i == 0)
    def _():
      pool_s[...] = p_part
      cnt_s[...] = c_part

    @pl.when(i > 0)
    def _():
      pool_s[...] = pool_s[...] + p_part
      cnt_s[...] = cnt_s[...] + c_part

    @pl.when(i == nb - 1)
    def _():
      pooled = pool_s[...] / jnp.maximum(cnt_s[...], 1.0)
      o_ref[...] = (jnp.dot(pooled, wlin_ref[...],
                            preferred_element_type=jnp.float32)
                    + blin_ref[...])

  return pl.pallas_call(
      body,
      grid=(nb,),
      in_specs=[
          pl.BlockSpec((NC, row_blk, d), lambda i: (0, i, 0)),
          pl.BlockSpec((1, NC, row_blk), lambda i: (i, 0, 0)),
          pl.BlockSpec((row_blk, d), lambda i: (i, 0)),
          pl.BlockSpec((d, d), lambda i: (0, 0)),
          pl.BlockSpec((d, d), lambda i: (0, 0)),
          pl.BlockSpec((1, d), lambda i: (0, 0)),
          pl.BlockSpec((1, 1, row_blk), lambda i: (i, 0, 0)),
          pl.BlockSpec((d, d), lambda i: (0, 0)),
          pl.BlockSpec((1, d), lambda i: (0, 0)),
      ],
      out_specs=pl.BlockSpec((n_graphs, d), lambda i: (0, 0)),
      out_shape=jax.ShapeDtypeStruct((n_graphs, d), jnp.float32),
      scratch_shapes=[
          pltpu.VMEM((n_graphs, d), jnp.float32),
          pltpu.VMEM((n_graphs, d), jnp.float32),
      ],
  )(acc, deg3, h1, wl, wr, b, bat3, wlin_p, blin_p)


def kernel(x, edge_index, batch, W0l, W0r, b0, W1l, W1r, b1, Wlin, blin):
  n, d = x.shape
  e = edge_index.shape[1]
  n_cls = Wlin.shape[1]
  n_graphs = 64
  row_blk = 400
  nb = n // row_blk

  grain = NW * CHUNK * NBUF
  e_pad = ((e + grain - 1) // grain) * grain
  n_pad = ((n + 1 + CHUNK - 1) // CHUNK) * CHUNK
  t_steps = e_pad // (NW * CHUNK)

  src = edge_index[0]
  dst = edge_index[1]
  pad = e_pad - e
  if pad:
    src = jnp.concatenate([src, jnp.zeros((pad,), jnp.int32)])
    dst = jnp.concatenate([dst, jnp.full((pad,), n, jnp.int32)])
  src = src.reshape(NW, t_steps, CHUNK)
  dst = dst.reshape(NW, t_steps, CHUNK)
  z2d = jnp.zeros((CHUNK, d), jnp.float32)
  ones1 = jnp.ones((CHUNK,), jnp.float32)

  acc0, deg = _make_sc_agg(n_pad, d, e_pad, True)(x, src, dst, z2d, ones1)
  deg3 = deg[:, :n].reshape(NC, nb, row_blk).transpose(1, 0, 2)

  b0r = b0.reshape(1, d)
  b1r = b1.reshape(1, d)
  h1 = _tc_layer(acc0, deg3, x, W0l, W0r, b0r, row_blk)

  (acc1,) = _make_sc_agg(n_pad, d, e_pad, False)(h1, src, dst, z2d, ones1)

  bat3 = batch.reshape(nb, 1, row_blk)
  wlin_p = jnp.zeros((d, d), jnp.float32).at[:, :n_cls].set(Wlin)
  blin_p = jnp.zeros((1, d), jnp.float32).at[0, :n_cls].set(blin)

  logits = _tc_final(acc1, deg3, h1, W1l, W1r, b1r, bat3, wlin_p, blin_p,
                     n_graphs, row_blk)
  return logits[:, :n_cls]


# speedup vs baseline: 1.6083x; 1.6083x over previous
import jax
import jax.numpy as jnp
from jax import lax
from jax.experimental import pallas as pl
from jax.experimental.pallas import tpu as pltpu
from jax.experimental.pallas import tpu_sc as plsc

NC = 2
NS = 16
CHUNK = 128
NBUF = 4


def _make_sc_agg(n, n_pad, d2, e_pad, with_deg):
  t_steps = e_pad // (NS * CHUNK)
  assert t_steps % NBUF == 0
  rowch = n_pad // CHUNK
  mesh = plsc.VectorSubcoreMesh(
      core_axis_name="c", subcore_axis_name="s",
      num_cores=NC, num_subcores=NS)

  out_type = [jax.ShapeDtypeStruct((NC, n_pad, d2), jnp.float32)]
  if with_deg:
    out_type.append(jax.ShapeDtypeStruct((n_pad,), jnp.float32))

  scratch = [
      [pltpu.VMEM((CHUNK,), jnp.int32) for _ in range(2 * NBUF)],
      [pltpu.VMEM((CHUNK, d2), jnp.float32) for _ in range(NBUF)],
      pltpu.VMEM((CHUNK,), jnp.float32),
      pltpu.VMEM_SHARED((n_pad, d2), jnp.float32),
      pltpu.VMEM_SHARED((n_pad,), jnp.float32),
      [pltpu.SemaphoreType.DMA for _ in range(NBUF)],
      [pltpu.SemaphoreType.DMA for _ in range(NBUF)],
      pltpu.SemaphoreType.DMA,
  ]

  def body(feats, srci, dsti, z2d, ones_in, *rest):
    if with_deg:
      acc_out, deg_out = rest[0], rest[1]
      rest = rest[2:]
    else:
      acc_out = rest[0]
      deg_out = None
      rest = rest[1:]
    ibufs, rows, ones_v, acc_sh, deg_sh, isems, rsems, dsem = rest
    sbuf = ibufs[:NBUF]
    dbuf = ibufs[NBUF:]

    c = lax.axis_index("c")
    s = lax.axis_index("s")

    pltpu.sync_copy(ones_in, ones_v)
    pltpu.sync_copy(z2d, rows[0])
    for jj in range((rowch + NS - 1) // NS):
      j = jj * NS + s

      @pl.when(j < rowch)
      def _():
        pltpu.sync_copy(rows[0], acc_sh.at[pl.ds(j * CHUNK, CHUNK)])
        if with_deg:
          @pl.when(c == 0)
          def _():
            pltpu.sync_copy(rows[0].at[0],
                            deg_sh.at[pl.ds(j * CHUNK, d2)])
            pltpu.sync_copy(rows[0].at[0],
                            deg_sh.at[pl.ds(j * CHUNK + d2, d2)])

    plsc.subcore_barrier()

    def idx_fetch(t, i):
      pltpu.async_copy(srci.at[c, s, t], sbuf[i], isems[i])
      pltpu.async_copy(dsti.at[s, t], dbuf[i], isems[i])

    def idx_wait(t, i):
      pltpu.make_async_copy(srci.at[c, s, t], sbuf[i], isems[i]).wait()
      pltpu.make_async_copy(dsti.at[s, t], dbuf[i], isems[i]).wait()

    def gather_start(i, r):
      pltpu.async_copy(feats.at[sbuf[i]], rows[r], rsems[r])

    def gather_wait(i, r):
      pltpu.make_async_copy(feats.at[sbuf[i]], rows[r], rsems[r]).wait()

    def deg_fire(i):
      pltpu.async_copy(ones_v, deg_sh.at[dbuf[i]], dsem, add=True)

    def deg_wait(i):
      pltpu.make_async_copy(ones_v, deg_sh.at[dbuf[i]], dsem).wait()

    for t in range(NBUF - 1):
      idx_fetch(t, t)
    idx_wait(0, 0)
    gather_start(0, 0)
    idx_wait(1, 1)
    gather_start(1, 1)

    def step(k, carry):
      for b in range(NBUF):
        t = NBUF * k + b

        if with_deg:
          @pl.when((c == 0) & (t > 0))
          def _():
            deg_wait((b - 1) % NBUF)

        @pl.when(t + NBUF - 1 < t_steps)
        def _():
          idx_fetch(t + NBUF - 1, (b + NBUF - 1) % NBUF)
        gather_wait(b, b)

        @pl.when(t + 2 < t_steps)
        def _():
          idx_wait(t + 2, (b + 2) % NBUF)
          gather_start((b + 2) % NBUF, (b + 2) % NBUF)
        pltpu.sync_copy(rows[b], acc_sh.at[dbuf[b]], add=True)
        if with_deg:
          @pl.when(c == 0)
          def _():
            deg_fire(b)
      return carry

    lax.fori_loop(0, t_steps // NBUF, step, 0)
    if with_deg:
      @pl.when(c == 0)
      def _():
        deg_wait((t_steps - 1) % NBUF)

    plsc.subcore_barrier()

    for jj in range((rowch + NS - 1) // NS):
      j = jj * NS + s

      @pl.when(j < rowch)
      def _():
        pltpu.sync_copy(acc_sh.at[pl.ds(j * CHUNK, CHUNK)],
                        acc_out.at[c, pl.ds(j * CHUNK, CHUNK)])
        if with_deg:
          @pl.when(c == 0)
          def _():
            pltpu.sync_copy(deg_sh.at[pl.ds(j * CHUNK, CHUNK)],
                            deg_out.at[pl.ds(j * CHUNK, CHUNK)])

  return pl.kernel(
      body, out_type=out_type, mesh=mesh, scratch_types=scratch,
      compiler_params=pltpu.CompilerParams(use_tc_tiling_on_sc=False))


def _tc_layer(acc, deg3, xin, wl, wr, b, row_blk):
  n, d = xin.shape
  d2 = d // 2
  nb = n // row_blk

  def body(acc_ref, deg_ref, x_ref, wl_ref, wr_ref, b_ref, o_ref):
    dtot = jnp.maximum(deg_ref[0, 0], 1.0)
    agg = jnp.concatenate([acc_ref[0], acc_ref[1]], axis=-1) / dtot[:, None]
    h = (jnp.dot(agg, wl_ref[...], preferred_element_type=jnp.float32)
         + jnp.dot(x_ref[...], wr_ref[...], preferred_element_type=jnp.float32)
         + b_ref[...])
    h = jnp.maximum(h, 0.0)
    o_ref[0] = h[:, :d2]
    o_ref[1] = h[:, d2:]

  return pl.pallas_call(
      body,
      grid=(nb,),
      in_specs=[
          pl.BlockSpec((NC, row_blk, d2), lambda i: (0, i, 0)),
          pl.BlockSpec((1, 1, row_blk), lambda i: (i, 0, 0)),
          pl.BlockSpec((row_blk, d), lambda i: (i, 0)),
          pl.BlockSpec((d, d), lambda i: (0, 0)),
          pl.BlockSpec((d, d), lambda i: (0, 0)),
          pl.BlockSpec((1, d), lambda i: (0, 0)),
      ],
      out_specs=pl.BlockSpec((NC, row_blk, d2), lambda i: (0, i, 0)),
      out_shape=jax.ShapeDtypeStruct((NC, n, d2), jnp.float32),
  )(acc, deg3, xin, wl, wr, b)


def _tc_final(acc, deg3, h1s, wl, wr, b, bat3, wlin_p, blin_p, n_graphs,
              row_blk):
  nc, n, d2 = h1s.shape
  d = d2 * 2
  nb = n // row_blk

  def body(acc_ref, deg_ref, h1_ref, wl_ref, wr_ref, b_ref, bat_ref,
           wlin_ref, blin_ref, o_ref, pool_s, cnt_s):
    i = pl.program_id(0)
    dtot = jnp.maximum(deg_ref[0, 0], 1.0)
    agg = jnp.concatenate([acc_ref[0], acc_ref[1]], axis=-1) / dtot[:, None]
    xin = jnp.concatenate([h1_ref[0], h1_ref[1]], axis=-1)
    h2 = jnp.maximum(
        jnp.dot(agg, wl_ref[...], preferred_element_type=jnp.float32)
        + jnp.dot(xin, wr_ref[...], preferred_element_type=jnp.float32)
        + b_ref[...], 0.0)

    bat = bat_ref[0, 0, :]
    seg = lax.broadcasted_iota(jnp.int32, (n_graphs, row_blk), 0)
    m = (seg == bat[None, :]).astype(jnp.float32)
    p_part = jnp.dot(m, h2, preferred_element_type=jnp.float32)
    c_part = jnp.broadcast_to(
        jnp.sum(m, axis=1, keepdims=True), (n_graphs, d))

    @pl.when(i == 0)
    def _():
      pool_s[...] = p_part
      cnt_s[...] = c_part

    @pl.when(i > 0)
    def _():
      pool_s[...] = pool_s[...] + p_part
      cnt_s[...] = cnt_s[...] + c_part

    @pl.when(i == nb - 1)
    def _():
      pooled = pool_s[...] / jnp.maximum(cnt_s[...], 1.0)
      o_ref[...] = (jnp.dot(pooled, wlin_ref[...],
                            preferred_element_type=jnp.float32)
                    + blin_ref[...])

  return pl.pallas_call(
      body,
      grid=(nb,),
      in_specs=[
          pl.BlockSpec((NC, row_blk, d2), lambda i: (0, i, 0)),
          pl.BlockSpec((1, 1, row_blk), lambda i: (i, 0, 0)),
          pl.BlockSpec((NC, row_blk, d2), lambda i: (0, i, 0)),
          pl.BlockSpec((d, d), lambda i: (0, 0)),
          pl.BlockSpec((d, d), lambda i: (0, 0)),
          pl.BlockSpec((1, d), lambda i: (0, 0)),
          pl.BlockSpec((1, 1, row_blk), lambda i: (i, 0, 0)),
          pl.BlockSpec((d, d), lambda i: (0, 0)),
          pl.BlockSpec((1, d), lambda i: (0, 0)),
      ],
      out_specs=pl.BlockSpec((n_graphs, d), lambda i: (0, 0)),
      out_shape=jax.ShapeDtypeStruct((n_graphs, d), jnp.float32),
      scratch_shapes=[
          pltpu.VMEM((n_graphs, d), jnp.float32),
          pltpu.VMEM((n_graphs, d), jnp.float32),
      ],
  )(acc, deg3, h1s, wl, wr, b, bat3, wlin_p, blin_p)


def kernel(x, edge_index, batch, W0l, W0r, b0, W1l, W1r, b1, Wlin, blin):
  n, d = x.shape
  d2 = d // 2
  e = edge_index.shape[1]
  n_cls = Wlin.shape[1]
  n_graphs = 64
  row_blk = 400
  nb = n // row_blk

  grain = NS * CHUNK * NBUF
  e_pad = ((e + grain - 1) // grain) * grain
  n_pad = ((n + 1 + CHUNK - 1) // CHUNK) * CHUNK
  t_steps = e_pad // (NS * CHUNK)

  src = edge_index[0]
  dst = edge_index[1]
  pad = e_pad - e
  if pad:
    src = jnp.concatenate([src, jnp.zeros((pad,), jnp.int32)])
    dst = jnp.concatenate([dst, jnp.full((pad,), n, jnp.int32)])
  src = src.reshape(NS, t_steps, CHUNK)
  dst = dst.reshape(NS, t_steps, CHUNK)
  z2d = jnp.zeros((CHUNK, d2), jnp.float32)
  ones1 = jnp.ones((CHUNK,), jnp.float32)

  src2 = jnp.stack([src, src + n])
  xh = x.reshape(n, NC, d2).transpose(1, 0, 2).reshape(NC * n, d2)
  acc0, deg = _make_sc_agg(n, n_pad, d2, e_pad, True)(
      xh, src2, dst, z2d, ones1)
  deg3 = deg[:n].reshape(nb, 1, row_blk)

  b0r = b0.reshape(1, d)
  b1r = b1.reshape(1, d)
  h1s = _tc_layer(acc0, deg3, x, W0l, W0r, b0r, row_blk)

  (acc1,) = _make_sc_agg(n, n_pad, d2, e_pad, False)(
      h1s.reshape(NC * n, d2), src2, dst, z2d, ones1)

  bat3 = batch.reshape(nb, 1, row_blk)
  wlin_p = jnp.zeros((d, d), jnp.float32).at[:, :n_cls].set(Wlin)
  blin_p = jnp.zeros((1, d), jnp.float32).at[0, :n_cls].set(blin)

  logits = _tc_final(acc1, deg3, h1s, W1l, W1r, b1r, bat3, wlin_p, blin_p,
                     n_graphs, row_blk)
  return logits[:, :n_cls]
